# baseline (speedup 1.0000x reference)
import functools

import jax
import jax.numpy as jnp
import numpy as np
from jax import lax
from jax.experimental import pallas as pl
from jax.experimental.pallas import tpu as pltpu
from jax.experimental.pallas import tpu_sc as plsc

_NUM_CAMERAS = 1024
_V = 10
_B = 16384
_U_GRID = [float(x) for x in np.linspace(0.0, 1.0, _V).astype(np.float32)]


def _so3_exp_c(phi):
    p0, p1, p2 = phi
    t2 = p0 * p0 + p1 * p1 + p2 * p2
    small = t2 < 1e-12
    th = jnp.sqrt(jnp.where(small, 1.0, t2))
    half = 0.5 * th
    k = jnp.where(small, 0.5 - t2 / 48.0, jnp.sin(half) / th)
    w = jnp.where(small, 1.0 - t2 / 8.0, jnp.cos(half))
    return [k * p0, k * p1, k * p2, w]


def _se3_exp_c(rho, phi):
    r0, r1, r2 = rho
    p0, p1, p2 = phi
    q = _so3_exp_c(phi)
    t2 = p0 * p0 + p1 * p1 + p2 * p2
    small = t2 < 1e-12
    s2 = jnp.where(small, 1.0, t2)
    th = jnp.sqrt(s2)
    a = jnp.where(small, 0.5 - t2 / 24.0, (1.0 - jnp.cos(th)) / s2)
    b = jnp.where(small, 1.0 / 6.0 - t2 / 120.0, (th - jnp.sin(th)) / (s2 * th))
    dot = p0 * r0 + p1 * r1 + p2 * r2
    cx = p1 * r2 - p2 * r1
    cy = p2 * r0 - p0 * r2
    cz = p0 * r1 - p1 * r0
    t = [r0 + a * cx + b * (p0 * dot - t2 * r0),
         r1 + a * cy + b * (p1 * dot - t2 * r1),
         r2 + a * cz + b * (p2 * dot - t2 * r2)]
    return t, q


def _qmul_c(q1, q2):
    x1, y1, z1, w1 = q1
    x2, y2, z2, w2 = q2
    return [w1 * x2 + x1 * w2 + y1 * z2 - z1 * y2,
            w1 * y2 - x1 * z2 + y1 * w2 + z1 * x2,
            w1 * z2 + x1 * y2 - y1 * x2 + z1 * w2,
            w1 * w2 - x1 * x2 - y1 * y2 - z1 * z2]


def _interp_body(p_ref, f_ref, o_ref):
    flag = f_ref[0]
    c = [p_ref[i] for i in range(12)]
    t0, q0 = _se3_exp_c(c[0:3], c[3:6])
    t1, q1 = _se3_exp_c(c[6:9], c[9:12])
    qi = [-q0[0], -q0[1], -q0[2], q0[3]]
    x, y, z, w = _qmul_c(qi, q1)
    n2 = x * x + y * y + z * z
    small = n2 < 1e-12
    n = jnp.sqrt(jnp.where(small, 1.0, n2))
    angle = 2.0 * jnp.arctan2(n, w)
    kl = jnp.where(small, 2.0 / w - 2.0 * n2 / (3.0 * (w * w * w)), angle / n)
    r = [kl * x, kl * y, kl * z]
    for v in range(_V):
        u = _U_GRID[v]
        tv = [(1.0 - u) * t0[i] + u * t1[i] for i in range(3)]
        qe = _so3_exp_c([u * r[0], u * r[1], u * r[2]])
        qv = _qmul_c(q0, qe)
        for i in range(3):
            o_ref[7 * v + i] = flag * tv[i]
        for i in range(4):
            o_ref[7 * v + 3 + i] = flag * qv[i]


def _interp_table(comp, flag):
    return pl.pallas_call(
        _interp_body,
        in_specs=[pl.BlockSpec(memory_space=pltpu.VMEM),
                  pl.BlockSpec(memory_space=pltpu.SMEM)],
        out_shape=jax.ShapeDtypeStruct((7 * _V, 8, 128), jnp.float32),
    )(comp, flag)


_BPW = _B // 32
_HPW = _B // 16


@functools.cache
def _make_sc_gather():
  mesh = plsc.VectorSubcoreMesh(core_axis_name="c", subcore_axis_name="s")

  @functools.partial(
    pl.kernel,
    out_type=jax.ShapeDtypeStruct((_V, _B // 128, 8, 128), jnp.float32),
    mesh=mesh,
    compiler_params=pltpu.CompilerParams(use_tc_tiling_on_sc=False,
                                         needs_layout_passes=False,
                                         disable_bounds_checks=True),
    scratch_types=[
        pltpu.VMEM((8, 128), jnp.int32),
        pltpu.VMEM((128,), jnp.int32),
        pltpu.VMEM((128,), jnp.int32),
        pltpu.VMEM((_NUM_CAMERAS,), jnp.int32),
        pltpu.VMEM((_NUM_CAMERAS,), jnp.int32),
        pltpu.VMEM((_NUM_CAMERAS,), jnp.int32),
        pltpu.VMEM((4, 128), jnp.int32),
        pltpu.VMEM((70 * _NUM_CAMERAS,), jnp.float32),
        pltpu.VMEM((70, 4, 128), jnp.float32),
        pltpu.VMEM_SHARED((_NUM_CAMERAS,), jnp.int32),
        pltpu.VMEM_SHARED((_NUM_CAMERAS,), jnp.int32),
        pltpu.SemaphoreType.DMA,
        pltpu.SemaphoreType.DMA,
    ],
  )
  def _sc_gather(idx_hbm, table_hbm, out_hbm, hidx_v, ones_v, zeros_v, hist_v,
                 u_v, remap_v, idx_v, tab_v, outbuf_v, sh_hist,
                 sh_remap, sem, sem2):
    cid = lax.axis_index("c")
    sid = lax.axis_index("s")
    wid = cid * 16 + sid
    ones16 = jnp.ones((16,), jnp.int32)
    zeros16 = jnp.zeros((16,), jnp.int32)
    for i in range(8):
        ones_v[pl.ds(i * 16, 16)] = ones16
        zeros_v[pl.ds(i * 16, 16)] = zeros16

    hloads = [pltpu.async_copy(idx_hbm.at[pl.ds(sid * _HPW + j * 128, 128)],
                               hidx_v.at[j], sem) for j in range(8)]
    base = wid * _BPW
    iloads = [pltpu.async_copy(idx_hbm.at[pl.ds(base + j * 128, 128)],
                               idx_v.at[j], sem) for j in range(4)]
    _TCH = 35 * _NUM_CAMERAS
    tloads = [pltpu.async_copy(table_hbm.at[pl.ds(j * _TCH, _TCH)],
                               tab_v.at[pl.ds(j * _TCH, _TCH)], sem2)
              for j in range(2)]

    @pl.when(sid == 0)
    def _zero_hist():
        for j in range(8):
            pltpu.sync_copy(zeros_v, sh_hist.at[pl.ds(j * 128, 128)])

    for h in hloads:
        h.wait()
    plsc.subcore_barrier()

    hadds = [pltpu.async_copy(ones_v, sh_hist.at[hidx_v.at[j]], sem, add=True)
             for j in range(8)]
    for h in hadds:
        h.wait()
    plsc.subcore_barrier()

    @pl.when(sid == 0)
    def _compact():
        pltpu.sync_copy(sh_hist, hist_v)
        iota16 = lax.iota(jnp.int32, 16)

        def cnt(j, carry):
            h = hist_v[pl.ds(j * 16, 16)]
            return carry + plsc.all_reduce_population_count(h > 0)

        nu_vec = lax.fori_loop(0, 64, cnt, jnp.zeros((16,), jnp.int32))
        nu = jnp.max(nu_vec)

        @pl.when(nu == _NUM_CAMERAS)
        def _identity():
            for j in range(64):
                remap_v[pl.ds(j * 16, 16)] = iota16 + (j * 16)

        @pl.when(nu != _NUM_CAMERAS)
        def _full():
            def chunk(j, carry):
                h = hist_v[pl.ds(j * 16, 16)]
                pm = h > 0
                p = pm.astype(jnp.int32)
                cs = plsc.cumsum(p)
                pos = cs + (carry - 1)
                cam = iota16 + j * 16
                plsc.store_scatter(u_v, [pos], cam, mask=pm)
                return carry + jnp.sum(p)

            lax.fori_loop(0, 64, chunk, jnp.int32(0))
            nm1 = nu - 1

            def chunk2(j, carry):
                cam = iota16 + j * 16
                cl = jnp.minimum(cam, nm1)
                remap_v[pl.ds(j * 16, 16)] = plsc.load_gather(u_v, [cl])
                return carry

            lax.fori_loop(0, 64, chunk2, jnp.int32(0))

        pltpu.sync_copy(remap_v, sh_remap)

    plsc.subcore_barrier()

    @pl.when(sid != 0)
    def _fetch_remap():
        pltpu.sync_copy(sh_remap, remap_v)

    for h in iloads:
        h.wait()
    rs = tuple(
        plsc.load_gather(remap_v,
                         [idx_v.at[k // 8][pl.ds((k % 8) * 16, 16)]])
        for k in range(32))

    def body(c, carry):
        for k in range(32):
            vals = plsc.load_gather(tab_v, [carry[k]])
            outbuf_v[c, k // 8, pl.ds((k % 8) * 16, 16)] = vals
        return tuple(v + _NUM_CAMERAS for v in carry)

    outs = []
    for j in range(2):
        tloads[j].wait()
        rs = plsc.parallel_loop(35 * j, 35 * (j + 1), unroll=5, carry=rs)(body)
        outs.extend(
            pltpu.async_copy(outbuf_v.at[t],
                             out_hbm.at[t // 7, pl.ds(wid * 4, 4), t % 7],
                             sem)
            for t in range(35 * j, 35 * (j + 1)))
    for h in outs:
        h.wait()

  return _sc_gather


def kernel(indices, is_training, pose_adjustment):
    indices = indices.astype(jnp.int32)
    comp = pose_adjustment.astype(jnp.float32).transpose(1, 2, 0).reshape(12, 8, 128)
    flag = jnp.asarray(is_training).astype(jnp.float32).reshape(1)
    table = _interp_table(comp, flag)
    out4 = _make_sc_gather()(indices, table.reshape(7 * _V * _NUM_CAMERAS))
    return out4.transpose(1, 3, 0, 2).reshape(_B, _V, 8)[:, :, :7]


# speedup vs baseline: 1.0117x; 1.0117x over previous
import functools

import jax
import jax.numpy as jnp
import numpy as np
from jax import lax
from jax.experimental import pallas as pl
from jax.experimental.pallas import tpu as pltpu
from jax.experimental.pallas import tpu_sc as plsc

_NUM_CAMERAS = 1024
_V = 10
_B = 16384
_U_GRID = [float(x) for x in np.linspace(0.0, 1.0, _V).astype(np.float32)]


def _so3_exp_c(phi):
    p0, p1, p2 = phi
    t2 = p0 * p0 + p1 * p1 + p2 * p2
    small = t2 < 1e-12
    th = jnp.sqrt(jnp.where(small, 1.0, t2))
    half = 0.5 * th
    k = jnp.where(small, 0.5 - t2 / 48.0, jnp.sin(half) / th)
    w = jnp.where(small, 1.0 - t2 / 8.0, jnp.cos(half))
    return [k * p0, k * p1, k * p2, w]


def _se3_exp_c(rho, phi):
    r0, r1, r2 = rho
    p0, p1, p2 = phi
    q = _so3_exp_c(phi)
    t2 = p0 * p0 + p1 * p1 + p2 * p2
    small = t2 < 1e-12
    s2 = jnp.where(small, 1.0, t2)
    th = jnp.sqrt(s2)
    a = jnp.where(small, 0.5 - t2 / 24.0, (1.0 - jnp.cos(th)) / s2)
    b = jnp.where(small, 1.0 / 6.0 - t2 / 120.0, (th - jnp.sin(th)) / (s2 * th))
    dot = p0 * r0 + p1 * r1 + p2 * r2
    cx = p1 * r2 - p2 * r1
    cy = p2 * r0 - p0 * r2
    cz = p0 * r1 - p1 * r0
    t = [r0 + a * cx + b * (p0 * dot - t2 * r0),
         r1 + a * cy + b * (p1 * dot - t2 * r1),
         r2 + a * cz + b * (p2 * dot - t2 * r2)]
    return t, q


def _qmul_c(q1, q2):
    x1, y1, z1, w1 = q1
    x2, y2, z2, w2 = q2
    return [w1 * x2 + x1 * w2 + y1 * z2 - z1 * y2,
            w1 * y2 - x1 * z2 + y1 * w2 + z1 * x2,
            w1 * z2 + x1 * y2 - y1 * x2 + z1 * w2,
            w1 * w2 - x1 * x2 - y1 * y2 - z1 * z2]


def _interp_body(p_ref, f_ref, o_ref):
    flag = f_ref[0]
    c = [p_ref[i] for i in range(12)]
    t0, q0 = _se3_exp_c(c[0:3], c[3:6])
    t1, q1 = _se3_exp_c(c[6:9], c[9:12])
    qi = [-q0[0], -q0[1], -q0[2], q0[3]]
    x, y, z, w = _qmul_c(qi, q1)
    n2 = x * x + y * y + z * z
    small = n2 < 1e-12
    n = jnp.sqrt(jnp.where(small, 1.0, n2))
    angle = 2.0 * jnp.arctan2(n, w)
    kl = jnp.where(small, 2.0 / w - 2.0 * n2 / (3.0 * (w * w * w)), angle / n)
    r = [kl * x, kl * y, kl * z]
    for v in range(_V):
        u = _U_GRID[v]
        tv = [(1.0 - u) * t0[i] + u * t1[i] for i in range(3)]
        qe = _so3_exp_c([u * r[0], u * r[1], u * r[2]])
        qv = _qmul_c(q0, qe)
        for i in range(3):
            o_ref[7 * v + i] = flag * tv[i]
        for i in range(4):
            o_ref[7 * v + 3 + i] = flag * qv[i]


def _interp_table(comp, flag):
    return pl.pallas_call(
        _interp_body,
        in_specs=[pl.BlockSpec(memory_space=pltpu.VMEM),
                  pl.BlockSpec(memory_space=pltpu.SMEM)],
        out_shape=jax.ShapeDtypeStruct((7 * _V, 8, 128), jnp.float32),
    )(comp, flag)


_BPW = _B // 32
_HPW = _B // 16


@functools.cache
def _make_sc_gather():
  mesh = plsc.VectorSubcoreMesh(core_axis_name="c", subcore_axis_name="s")

  @functools.partial(
    pl.kernel,
    out_type=jax.ShapeDtypeStruct((_V, _B // 128, 8, 128), jnp.float32),
    mesh=mesh,
    compiler_params=pltpu.CompilerParams(use_tc_tiling_on_sc=False,
                                         needs_layout_passes=False,
                                         disable_bounds_checks=True),
    scratch_types=[
        pltpu.VMEM((8, 128), jnp.int32),
        pltpu.VMEM((128,), jnp.int32),
        pltpu.VMEM((128,), jnp.int32),
        pltpu.VMEM((_NUM_CAMERAS,), jnp.int32),
        pltpu.VMEM((_NUM_CAMERAS,), jnp.int32),
        pltpu.VMEM((_NUM_CAMERAS,), jnp.int32),
        pltpu.VMEM((4, 128), jnp.int32),
        pltpu.VMEM((70 * _NUM_CAMERAS,), jnp.float32),
        pltpu.VMEM((70, 4, 128), jnp.float32),
        pltpu.VMEM_SHARED((_NUM_CAMERAS,), jnp.int32),
        pltpu.VMEM_SHARED((_NUM_CAMERAS,), jnp.int32),
        pltpu.SemaphoreType.DMA,
        pltpu.SemaphoreType.DMA,
    ],
  )
  def _sc_gather(idx_hbm, table_hbm, out_hbm, hidx_v, ones_v, zeros_v, hist_v,
                 u_v, remap_v, idx_v, tab_v, outbuf_v, sh_hist,
                 sh_remap, sem, sem2):
    cid = lax.axis_index("c")
    sid = lax.axis_index("s")
    wid = cid * 16 + sid
    ones16 = jnp.ones((16,), jnp.int32)
    zeros16 = jnp.zeros((16,), jnp.int32)
    for i in range(8):
        ones_v[pl.ds(i * 16, 16)] = ones16
        zeros_v[pl.ds(i * 16, 16)] = zeros16

    hloads = [pltpu.async_copy(idx_hbm.at[pl.ds(sid * _HPW + j * 128, 128)],
                               hidx_v.at[j], sem) for j in range(8)]
    base = wid * _BPW
    iloads = [pltpu.async_copy(idx_hbm.at[pl.ds(base + j * 128, 128)],
                               idx_v.at[j], sem) for j in range(4)]
    _TCH = 35 * _NUM_CAMERAS
    tloads = [pltpu.async_copy(table_hbm.at[pl.ds(j * _TCH, _TCH)],
                               tab_v.at[pl.ds(j * _TCH, _TCH)], sem2)
              for j in range(2)]

    @pl.when(sid == 0)
    def _zero_hist():
        for j in range(8):
            pltpu.sync_copy(zeros_v, sh_hist.at[pl.ds(j * 128, 128)])

    for h in hloads:
        h.wait()
    plsc.subcore_barrier()

    for j in range(8):
        pltpu.sync_copy(ones_v, sh_hist.at[hidx_v.at[j]], add=True)
    plsc.subcore_barrier()

    @pl.when(sid == 0)
    def _compact():
        pltpu.sync_copy(sh_hist, hist_v)
        iota16 = lax.iota(jnp.int32, 16)

        def cnt(j, carry):
            h = hist_v[pl.ds(j * 16, 16)]
            return carry + plsc.all_reduce_population_count(h > 0)

        nu_vec = lax.fori_loop(0, 64, cnt, jnp.zeros((16,), jnp.int32))
        nu = jnp.max(nu_vec)

        @pl.when(nu == _NUM_CAMERAS)
        def _identity():
            for j in range(64):
                remap_v[pl.ds(j * 16, 16)] = iota16 + (j * 16)

        @pl.when(nu != _NUM_CAMERAS)
        def _full():
            def chunk(j, carry):
                h = hist_v[pl.ds(j * 16, 16)]
                pm = h > 0
                p = pm.astype(jnp.int32)
                cs = plsc.cumsum(p)
                pos = cs + (carry - 1)
                cam = iota16 + j * 16
                plsc.store_scatter(u_v, [pos], cam, mask=pm)
                return carry + jnp.sum(p)

            lax.fori_loop(0, 64, chunk, jnp.int32(0))
            nm1 = nu - 1

            def chunk2(j, carry):
                cam = iota16 + j * 16
                cl = jnp.minimum(cam, nm1)
                remap_v[pl.ds(j * 16, 16)] = plsc.load_gather(u_v, [cl])
                return carry

            lax.fori_loop(0, 64, chunk2, jnp.int32(0))

        pltpu.sync_copy(remap_v, sh_remap)

    plsc.subcore_barrier()

    @pl.when(sid != 0)
    def _fetch_remap():
        pltpu.sync_copy(sh_remap, remap_v)

    for h in iloads:
        h.wait()
    rs = tuple(
        plsc.load_gather(remap_v,
                         [idx_v.at[k // 8][pl.ds((k % 8) * 16, 16)]])
        for k in range(32))

    def body(c, carry):
        for k in range(32):
            vals = plsc.load_gather(tab_v, [carry[k]])
            outbuf_v[c, k // 8, pl.ds((k % 8) * 16, 16)] = vals
        return tuple(v + _NUM_CAMERAS for v in carry)

    outs = []
    for j in range(2):
        tloads[j].wait()
        rs = plsc.parallel_loop(35 * j, 35 * (j + 1), unroll=5, carry=rs)(body)
        outs.extend(
            pltpu.async_copy(outbuf_v.at[t],
                             out_hbm.at[t // 7, pl.ds(wid * 4, 4), t % 7],
                             sem)
            for t in range(35 * j, 35 * (j + 1)))
    for h in outs:
        h.wait()

  return _sc_gather


def kernel(indices, is_training, pose_adjustment):
    indices = indices.astype(jnp.int32)
    comp = pose_adjustment.astype(jnp.float32).transpose(1, 2, 0).reshape(12, 8, 128)
    flag = jnp.asarray(is_training).astype(jnp.float32).reshape(1)
    table = _interp_table(comp, flag)
    out4 = _make_sc_gather()(indices, table.reshape(7 * _V * _NUM_CAMERAS))
    return out4.transpose(1, 3, 0, 2).reshape(_B, _V, 8)[:, :, :7]
